# baseline (speedup 1.0000x reference)
import jax
import jax.numpy as jnp
from jax import lax
from jax.experimental import pallas as pl
from jax.experimental.pallas import tpu as pltpu
from jax.experimental.pallas import tpu_sc as plsc

DIM = 64
FIELDS = 26

NC = 2
NS = 16
NW = NC * NS

BT = 128
BT_PER_W = 4
NBLK = FIELDS * BT_PER_W


def _body(xT_hbm, table_hbm, out_hbm, idx_v, rows0, rows1, stg0, stg1,
          gsem0, gsem1, wsem0, wsem1):
    wid = lax.axis_index("s") * NC + lax.axis_index("c")
    b0 = wid * (BT_PER_W * BT)

    pltpu.sync_copy(xT_hbm.at[:, pl.ds(b0, BT_PER_W * BT)], idx_v)

    rows = (rows0, rows1)
    stg = (stg0, stg1)
    gsems = (gsem0, gsem1)
    wsems = (wsem0, wsem1)

    def fire_gather(f, btl, buf, sem):
        pltpu.async_copy(
            table_hbm.at[idx_v.at[f, pl.ds(btl * BT, BT)]], buf, sem)

    def wait_gather(buf, sem):
        pltpu.make_async_copy(table_hbm.at[pl.ds(0, BT)], buf, sem).wait()

    def fire_write(f, btl, buf, sem):
        pltpu.async_copy(buf, out_hbm.at[f, :, wid * BT_PER_W + btl], sem)

    def wait_write(buf, sem):
        pltpu.make_async_copy(out_hbm.at[0, :, 0], buf, sem).wait()

    iota = lax.iota(jnp.int32, 16)

    def transpose(rbuf, sbuf):
        def db_body(db, carry):
            for s in range(8):
                col = jnp.full((16,), db * 8 + s, jnp.int32)
                for j in range(8):
                    rid = iota + (j * 16)
                    vals = plsc.load_gather(rbuf, [rid, col])
                    sbuf[db, s, pl.ds(j * 16, 16)] = vals
            return carry
        lax.fori_loop(0, 8, db_body, 0)

    def step(f, btl, f2, btl2, b, first, last):
        if not first:
            wait_write(stg[b], wsems[b])
        wait_gather(rows[b], gsems[b])
        transpose(rows[b], stg[b])
        fire_write(f, btl, stg[b], wsems[b])
        if not last:
            fire_gather(f2, btl2, rows[b], gsems[b])

    def advance(f, btl):
        wrap = f == FIELDS - 1
        return (jnp.where(wrap, 0, f + 1),
                jnp.where(wrap, btl + 1, btl))

    fire_gather(0, 0, rows0, gsem0)
    fire_gather(1, 0, rows1, gsem1)

    step(0, 0, 2, 0, 0, True, False)
    step(1, 0, 3, 0, 1, True, False)

    def body(i, carry):
        f, btl = carry
        for b in range(2):
            f1, btl1 = advance(f, btl)
            f2, btl2 = advance(f1, btl1)
            step(f, btl, f2, btl2, b, False, False)
            f, btl = f1, btl1
        return (f, btl)

    lax.fori_loop(1, (NBLK - 2) // 2, body,
                  (jnp.int32(2), jnp.int32(0)))

    for k in (NBLK - 2, NBLK - 1):
        f, btl = k % FIELDS, k // FIELDS
        step(f, btl, 0, 0, k % 2, False, True)

    for b in range(2):
        wait_write(stg[b], wsems[b])


def kernel(x, table):
    batch, fields = x.shape
    xT = jnp.transpose(x)

    out5 = pl.kernel(
        _body,
        out_type=jax.ShapeDtypeStruct(
            (FIELDS, DIM // 8, batch // BT, 8, BT), jnp.float32),
        mesh=plsc.VectorSubcoreMesh(core_axis_name="c", subcore_axis_name="s"),
        compiler_params=pltpu.CompilerParams(
            use_tc_tiling_on_sc=False, needs_layout_passes=False),
        scratch_types=[
            pltpu.VMEM((FIELDS, BT_PER_W * BT), jnp.int32),
            pltpu.VMEM((BT, DIM), jnp.float32),
            pltpu.VMEM((BT, DIM), jnp.float32),
            pltpu.VMEM((8, 8, BT), jnp.float32),
            pltpu.VMEM((8, 8, BT), jnp.float32),
            pltpu.SemaphoreType.DMA,
            pltpu.SemaphoreType.DMA,
            pltpu.SemaphoreType.DMA,
            pltpu.SemaphoreType.DMA,
        ],
    )(xT, table)
    return jnp.transpose(out5, (2, 4, 0, 1, 3)).reshape(batch, fields, DIM)


# speedup vs baseline: 1.1141x; 1.1141x over previous
import jax
import jax.numpy as jnp
from jax import lax
from jax.experimental import pallas as pl
from jax.experimental.pallas import tpu as pltpu
from jax.experimental.pallas import tpu_sc as plsc

DIM = 64
FIELDS = 26

NC = 2
NS = 16
NW = NC * NS

BT = 128
BT_PER_W = 4
NBLK = FIELDS * BT_PER_W


def _body(xT_hbm, table_hbm, out_hbm, idx_v, rows0, rows1, stg0, stg1,
          gsem0, gsem1, wsem0, wsem1):
    wid = lax.axis_index("s") * NC + lax.axis_index("c")
    b0 = wid * (BT_PER_W * BT)

    pltpu.sync_copy(xT_hbm.at[:, pl.ds(b0, BT_PER_W * BT)], idx_v)

    rows = (rows0, rows1)
    stg = (stg0, stg1)
    gsems = (gsem0, gsem1)
    wsems = (wsem0, wsem1)

    def fire_gather(f, btl, buf, sem):
        pltpu.async_copy(
            table_hbm.at[idx_v.at[f, pl.ds(btl * BT, BT)]], buf, sem)

    def wait_gather(buf, sem):
        pltpu.make_async_copy(table_hbm.at[pl.ds(0, BT)], buf, sem).wait()

    def fire_write(f, btl, buf, sem):
        bt = wid * BT_PER_W + btl
        for db in range(8):
            pltpu.async_copy(buf.at[pl.ds(db * 8, 8)], out_hbm.at[f, db, bt], sem)

    def wait_write(buf, sem):
        d = pltpu.make_async_copy(out_hbm.at[0, 0, 0], buf.at[pl.ds(0, 8)], sem)
        for _ in range(8):
            d.wait()

    iota = lax.iota(jnp.int32, 16)
    dbases = [iota + c0 for c0 in range(0, DIM, 16)]
    one = jnp.full((16,), 1, jnp.int32)

    def transpose(rbuf, sbuf):
        vb = jnp.full((16,), 0, jnp.int32)
        for b in range(BT):
            for k in range(DIM // 16):
                vals = rbuf[b, pl.ds(k * 16, 16)]
                plsc.store_scatter(sbuf, [dbases[k], vb], vals)
            vb = vb + one

    def advance(f, btl):
        wrap = f == FIELDS - 1
        return (jnp.where(wrap, 0, f + 1),
                jnp.where(wrap, btl + 1, btl))

    fire_gather(0, 0, rows0, gsem0)
    fire_gather(1, 0, rows1, gsem1)

    def body(i, carry):
        f, btl = carry
        for b in range(2):
            k = i * 2 + b
            f1, btl1 = advance(f, btl)
            f2, btl2 = advance(f1, btl1)

            @pl.when(k >= 2)
            def _():
                wait_write(stg[b], wsems[b])

            wait_gather(rows[b], gsems[b])
            transpose(rows[b], stg[b])
            fire_write(f, btl, stg[b], wsems[b])

            @pl.when(k <= NBLK - 3)
            def _():
                fire_gather(f2, btl2, rows[b], gsems[b])

            f, btl = f1, btl1
        return (f, btl)

    lax.fori_loop(0, NBLK // 2, body, (jnp.int32(0), jnp.int32(0)))

    for b in range(2):
        wait_write(stg[b], wsems[b])


def kernel(x, table):
    batch, fields = x.shape
    xT = jnp.transpose(x)

    out5 = pl.kernel(
        _body,
        out_type=jax.ShapeDtypeStruct(
            (FIELDS, DIM // 8, batch // BT, 8, BT), jnp.float32),
        mesh=plsc.VectorSubcoreMesh(core_axis_name="c", subcore_axis_name="s"),
        compiler_params=pltpu.CompilerParams(
            use_tc_tiling_on_sc=False, needs_layout_passes=False),
        scratch_types=[
            pltpu.VMEM((FIELDS, BT_PER_W * BT), jnp.int32),
            pltpu.VMEM((BT, DIM), jnp.float32),
            pltpu.VMEM((BT, DIM), jnp.float32),
            pltpu.VMEM((DIM, BT), jnp.float32),
            pltpu.VMEM((DIM, BT), jnp.float32),
            pltpu.SemaphoreType.DMA,
            pltpu.SemaphoreType.DMA,
            pltpu.SemaphoreType.DMA,
            pltpu.SemaphoreType.DMA,
        ],
    )(xT, table)
    return jnp.transpose(out5, (2, 4, 0, 1, 3)).reshape(batch, fields, DIM)


# speedup vs baseline: 1.1267x; 1.0113x over previous
import jax
import jax.numpy as jnp
from jax import lax
from jax.experimental import pallas as pl
from jax.experimental.pallas import tpu as pltpu
from jax.experimental.pallas import tpu_sc as plsc

DIM = 64
FIELDS = 26

NC = 2
NS = 16
NW = NC * NS

BT = 128
BT_PER_W = 4
NBLK = FIELDS * BT_PER_W


def _body(xT_hbm, table_hbm, out_hbm, idx_v, rows0, rows1, stg0, stg1,
          gsem0, gsem1, wsem0, wsem1):
    wid = lax.axis_index("s") * NC + lax.axis_index("c")
    b0 = wid * (BT_PER_W * BT)

    pltpu.sync_copy(xT_hbm.at[:, pl.ds(b0, BT_PER_W * BT)], idx_v)

    rows = (rows0, rows1)
    stg = (stg0, stg1)
    gsems = (gsem0, gsem1)
    wsems = (wsem0, wsem1)

    def fire_gather(f, btl, buf, sem):
        pltpu.async_copy(
            table_hbm.at[idx_v.at[f, pl.ds(btl * BT, BT)]], buf, sem)

    def wait_gather(buf, sem):
        pltpu.make_async_copy(table_hbm.at[pl.ds(0, BT)], buf, sem).wait()

    def fire_write(f, btl, buf, sem):
        bt = wid * BT_PER_W + btl
        for db in range(8):
            pltpu.async_copy(buf.at[pl.ds(db * 8, 8)], out_hbm.at[f, db, bt], sem)

    def wait_write(buf, sem):
        d = pltpu.make_async_copy(out_hbm.at[0, 0, 0], buf.at[pl.ds(0, 8)], sem)
        for _ in range(8):
            d.wait()

    iota = lax.iota(jnp.int32, 16)
    dbases = [iota + c0 for c0 in range(0, DIM, 16)]
    one = jnp.full((16,), 1, jnp.int32)

    def transpose(rbuf, sbuf):
        nk = DIM // 16
        lead = 4

        def load_row(b):
            return [rbuf[b, pl.ds(k * 16, 16)] for k in range(nk)]

        pending = [load_row(b) for b in range(lead)]
        vb = jnp.full((16,), 0, jnp.int32)
        for b in range(BT):
            if b + lead < BT:
                pending.append(load_row(b + lead))
            vals = pending[b]
            for k in range(nk):
                plsc.store_scatter(sbuf, [dbases[k], vb], vals[k])
            vb = vb + one

    def advance(f, btl):
        wrap = f == FIELDS - 1
        return (jnp.where(wrap, 0, f + 1),
                jnp.where(wrap, btl + 1, btl))

    fire_gather(0, 0, rows0, gsem0)
    fire_gather(1, 0, rows1, gsem1)

    def body(i, carry):
        f, btl = carry
        for b in range(2):
            k = i * 2 + b
            f1, btl1 = advance(f, btl)
            f2, btl2 = advance(f1, btl1)

            @pl.when(k >= 2)
            def _():
                wait_write(stg[b], wsems[b])

            wait_gather(rows[b], gsems[b])
            transpose(rows[b], stg[b])
            fire_write(f, btl, stg[b], wsems[b])

            @pl.when(k <= NBLK - 3)
            def _():
                fire_gather(f2, btl2, rows[b], gsems[b])

            f, btl = f1, btl1
        return (f, btl)

    lax.fori_loop(0, NBLK // 2, body, (jnp.int32(0), jnp.int32(0)))

    for b in range(2):
        wait_write(stg[b], wsems[b])


def kernel(x, table):
    batch, fields = x.shape
    xT = jnp.transpose(x)

    out5 = pl.kernel(
        _body,
        out_type=jax.ShapeDtypeStruct(
            (FIELDS, DIM // 8, batch // BT, 8, BT), jnp.float32),
        mesh=plsc.VectorSubcoreMesh(core_axis_name="c", subcore_axis_name="s"),
        compiler_params=pltpu.CompilerParams(
            use_tc_tiling_on_sc=False, needs_layout_passes=False),
        scratch_types=[
            pltpu.VMEM((FIELDS, BT_PER_W * BT), jnp.int32),
            pltpu.VMEM((BT, DIM), jnp.float32),
            pltpu.VMEM((BT, DIM), jnp.float32),
            pltpu.VMEM((DIM, BT), jnp.float32),
            pltpu.VMEM((DIM, BT), jnp.float32),
            pltpu.SemaphoreType.DMA,
            pltpu.SemaphoreType.DMA,
            pltpu.SemaphoreType.DMA,
            pltpu.SemaphoreType.DMA,
        ],
    )(xT, table)
    return jnp.transpose(out5, (2, 4, 0, 1, 3)).reshape(batch, fields, DIM)


# speedup vs baseline: 1.3496x; 1.1978x over previous
import jax
import jax.numpy as jnp
from jax import lax
from jax.experimental import pallas as pl
from jax.experimental.pallas import tpu as pltpu
from jax.experimental.pallas import tpu_sc as plsc

DIM = 64

NC = 2
NS = 16
NW = NC * NS

IDXW = 128
GPC = 2
CHUNK = IDXW * GPC
WIDE = 128


def _gather_body(idx_hbm, table_hbm, out_hbm, idx_v, rows0, rows1, sem0, sem1):
    wid = lax.axis_index("s") * NC + lax.axis_index("c")
    rows_per_w = idx_hbm.shape[1] * idx_hbm.shape[2]
    n_chunks = rows_per_w // CHUNK
    base = wid * rows_per_w

    pltpu.sync_copy(idx_hbm.at[wid], idx_v)

    bufs = (rows0, rows1)
    sems = (sem0, sem1)

    def fire(c, buf, sem):
        for j in range(GPC):
            pltpu.async_copy(
                table_hbm.at[idx_v.at[c * GPC + j]],
                buf.at[pl.ds(j * IDXW, IDXW)],
                sem,
            )

    def drain(buf, sem):
        pltpu.make_async_copy(table_hbm.at[pl.ds(0, CHUNK)], buf, sem).wait()

    fire(0, rows0, sem0)
    fire(1, rows1, sem1)

    def body(i, carry):
        for b in range(2):
            c = i * 2 + b
            drain(bufs[b], sems[b])
            pltpu.sync_copy(bufs[b].at[:, pl.ds(0, DIM)],
                            out_hbm.at[pl.ds(base + c * CHUNK, CHUNK)])
            fire(c + 2, bufs[b], sems[b])
        return carry

    lax.fori_loop(0, (n_chunks - 2) // 2, body, 0)

    for b in range(2):
        c = n_chunks - 2 + b
        drain(bufs[b], sems[b])
        pltpu.sync_copy(bufs[b].at[:, pl.ds(0, DIM)],
                        out_hbm.at[pl.ds(base + c * CHUNK, CHUNK)])


def kernel(x, table):
    batch, fields = x.shape
    total = batch * fields
    rows_per_w = total // NW
    idx3 = x.reshape(NW, rows_per_w // IDXW, IDXW)
    table_wide = jnp.pad(table, ((0, 0), (0, 128 - DIM)))

    out = pl.kernel(
        _gather_body,
        out_type=jax.ShapeDtypeStruct((total, DIM), jnp.float32),
        mesh=plsc.VectorSubcoreMesh(core_axis_name="c", subcore_axis_name="s"),
        compiler_params=pltpu.CompilerParams(use_tc_tiling_on_sc=False),
        scratch_types=[
            pltpu.VMEM((rows_per_w // IDXW, IDXW), jnp.int32),
            pltpu.VMEM((CHUNK, WIDE), jnp.float32),
            pltpu.VMEM((CHUNK, WIDE), jnp.float32),
            pltpu.SemaphoreType.DMA,
            pltpu.SemaphoreType.DMA,
        ],
    )(idx3, table_wide)
    return out.reshape(batch, fields, DIM)
